# Initial kernel scaffold; baseline (speedup 1.0000x reference)
import jax
import jax.numpy as jnp
from jax.experimental import pallas as pl


def kernel(x, W_enc, b_enc, W_dec, b_dec):
    raise NotImplementedError("write your pallas kernel here")


# speedup vs baseline: 18.8200x; 18.8200x over previous
import functools

import jax
import jax.numpy as jnp
from jax import lax
from jax.experimental import pallas as pl
from jax.experimental.pallas import tpu as pltpu
from jax.experimental.pallas import tpu_sc as plsc

_NBUK = 65536
_LANES = 16


def _encode_body(x_ref, w_ref, be_ref, bd_ref, acts_ref):
    xm = x_ref[...] - bd_ref[...]
    a = lax.dot_general(xm, w_ref[...], (((1,), (1,)), ((), ())),
                        preferred_element_type=jnp.float32)
    acts_ref[...] = jnp.maximum(a + be_ref[...], 0.0)


def _encode(x, w_enc, b_enc, b_dec, ft):
    b, d = x.shape
    f = w_enc.shape[0]
    grid = (f // ft,)
    return pl.pallas_call(
        _encode_body,
        grid=grid,
        in_specs=[
            pl.BlockSpec((b, d), lambda i: (0, 0)),
            pl.BlockSpec((ft, d), lambda i: (i, 0)),
            pl.BlockSpec((1, ft), lambda i: (0, i)),
            pl.BlockSpec((1, d), lambda i: (0, 0)),
        ],
        out_specs=pl.BlockSpec((b, ft), lambda i: (0, i)),
        out_shape=jax.ShapeDtypeStruct((b, f), jnp.float32),
    )(x, w_enc, b_enc.reshape(1, f), b_dec.reshape(1, d))


def _make_hist(n, chunk):
    info = plsc.get_sparse_core_info()
    nw = info.num_cores * info.num_subcores
    per_w = n // nw
    n_chunks = per_w // chunk
    assert per_w % chunk == 0 and chunk % (8 * _LANES) == 0
    mesh = plsc.VectorSubcoreMesh(core_axis_name="c", subcore_axis_name="s")

    @functools.partial(
        pl.kernel,
        mesh=mesh,
        out_type=jax.ShapeDtypeStruct((nw, _NBUK), jnp.int32),
        compiler_params=pltpu.CompilerParams(needs_layout_passes=False),
        scratch_types=[
            pltpu.VMEM((chunk,), jnp.float32),
            pltpu.VMEM((chunk,), jnp.float32),
            pltpu.VMEM((_NBUK,), jnp.int32),
            pltpu.VMEM((3 * _LANES,), jnp.int32),
            pltpu.SemaphoreType.DMA,
            pltpu.SemaphoreType.DMA,
        ],
    )
    def hist_kernel(acts_hbm, params_hbm, out_hbm, buf0, buf1, hist_v, par_v,
                    sem0, sem1):
        wid = lax.axis_index("s") * info.num_cores + lax.axis_index("c")
        base = wid * per_w

        pltpu.sync_copy(params_hbm, par_v)
        shift = par_v[pl.ds(0, _LANES)]
        fmask = par_v[pl.ds(_LANES, _LANES)]
        fbits = par_v[pl.ds(2 * _LANES, _LANES)]

        ones = jnp.ones((_LANES,), jnp.int32)
        zeros = jnp.zeros((_LANES,), jnp.int32)
        lowmask = jnp.full((_LANES,), 0xFFFF, jnp.int32)

        def zero_body(i, carry):
            hist_v[pl.ds(i * _LANES, _LANES)] = zeros
            return carry

        lax.fori_loop(0, _NBUK // _LANES, zero_body, 0)

        def process(buf):
            def vbody(i, carry):
                for u in range(8):
                    v = buf[pl.ds((i * 8 + u) * _LANES, _LANES)]
                    bits = plsc.bitcast(v, jnp.int32)
                    match = ((bits & fmask) == fbits) & (bits != zeros)
                    buk = lax.shift_right_logical(bits, shift) & lowmask
                    plsc.addupdate_scatter(hist_v, [buk], ones, mask=match)
                return carry

            lax.fori_loop(0, chunk // _LANES // 8, vbody, 0)

        def copy(c, buf, sem):
            return pltpu.make_async_copy(
                acts_hbm.at[pl.ds(base + c * chunk, chunk)], buf, sem)

        copy(0, buf0, sem0).start()

        def cbody(i, carry):
            for par, (buf_a, sem_a, buf_b, sem_b) in (
                    (0, (buf0, sem0, buf1, sem1)),
                    (1, (buf1, sem1, buf0, sem0))):
                c = i * 2 + par

                @pl.when(c + 1 < n_chunks)
                def _():
                    copy(c + 1, buf_b, sem_b).start()

                copy(c, buf_a, sem_a).wait()
                process(buf_a)
            return carry

        lax.fori_loop(0, n_chunks // 2, cbody, 0)
        pltpu.sync_copy(hist_v, out_hbm.at[wid])

    return hist_kernel


def _bcast16(v):
    return jnp.full((_LANES,), v, jnp.int32)


def _rank_threshold(acts_flat, kb):
    n = acts_flat.shape[0]
    hist_fn = _make_hist(n, 8192)

    params_a = jnp.concatenate([_bcast16(16), _bcast16(0), _bcast16(0)])
    cnt_a = hist_fn(acts_flat, params_a).sum(axis=0)
    t_a = jnp.cumsum(cnt_a[::-1])[::-1]
    hstar = jnp.sum(t_a >= kb).astype(jnp.int32) - 1
    t_next = jnp.concatenate([t_a[1:], jnp.zeros((1,), t_a.dtype)])
    r = kb - t_next[jnp.maximum(hstar, 0)]

    hshift = jnp.left_shift(jnp.maximum(hstar, 0), 16)
    params_b = jnp.concatenate([
        _bcast16(0),
        _bcast16(-65536),
        jnp.full((_LANES,), hshift, jnp.int32),
    ])
    cnt_b = hist_fn(acts_flat, params_b).sum(axis=0)
    t_b = jnp.cumsum(cnt_b[::-1])[::-1]
    lstar = jnp.sum(t_b >= r).astype(jnp.int32) - 1

    vk_bits = hshift | jnp.maximum(lstar, 0)
    vk = lax.bitcast_convert_type(vk_bits, jnp.float32)
    return jnp.where(hstar < 0, jnp.float32(0.0), vk)


def _decode_body(vk_ref, acts_ref, w_ref, bd_ref, out_ref):
    i = pl.program_id(0)
    vk = vk_ref[0, 0]
    a = acts_ref[...]
    m = jnp.where(a >= vk, a, 0.0)
    part = lax.dot_general(m, w_ref[...], (((1,), (1,)), ((), ())),
                           preferred_element_type=jnp.float32)

    @pl.when(i == 0)
    def _():
        out_ref[...] = jnp.broadcast_to(bd_ref[...], out_ref.shape)

    out_ref[...] += part


def _decode(vk, acts, w_dec, b_dec, ft):
    b, f = acts.shape
    d = w_dec.shape[0]
    grid = (f // ft,)
    return pl.pallas_call(
        _decode_body,
        grid=grid,
        in_specs=[
            pl.BlockSpec(memory_space=pltpu.SMEM),
            pl.BlockSpec((b, ft), lambda i: (0, i)),
            pl.BlockSpec((d, ft), lambda i: (0, i)),
            pl.BlockSpec((1, d), lambda i: (0, 0)),
        ],
        out_specs=pl.BlockSpec((b, d), lambda i: (0, 0)),
        out_shape=jax.ShapeDtypeStruct((b, d), jnp.float32),
    )(vk.reshape(1, 1), acts, w_dec, b_dec.reshape(1, d))


def kernel(x, W_enc, b_enc, W_dec, b_dec):
    b, d = x.shape
    f = W_enc.shape[0]
    kb = min(64 * b, b * f)
    acts = _encode(x, W_enc, b_enc, b_dec, ft=2048)
    vk = _rank_threshold(acts.reshape(-1), kb)
    return _decode(vk, acts, W_dec, b_dec, ft=2048)


# speedup vs baseline: 20.6263x; 1.0960x over previous
import functools

import jax
import jax.numpy as jnp
from jax import lax
from jax.experimental import pallas as pl
from jax.experimental.pallas import tpu as pltpu
from jax.experimental.pallas import tpu_sc as plsc

_NBUK = 65536
_LANES = 16


def _encode_body(x_ref, w_ref, be_ref, bd_ref, acts_ref):
    xm = x_ref[...] - bd_ref[...]
    a = lax.dot_general(xm, w_ref[...], (((1,), (1,)), ((), ())),
                        preferred_element_type=jnp.float32)
    acts_ref[...] = jnp.maximum(a + be_ref[...], 0.0)


def _encode(x, w_enc, b_enc, b_dec, ft):
    b, d = x.shape
    f = w_enc.shape[0]
    grid = (f // ft,)
    return pl.pallas_call(
        _encode_body,
        grid=grid,
        in_specs=[
            pl.BlockSpec((b, d), lambda i: (0, 0)),
            pl.BlockSpec((ft, d), lambda i: (i, 0)),
            pl.BlockSpec((1, ft), lambda i: (0, i)),
            pl.BlockSpec((1, d), lambda i: (0, 0)),
        ],
        out_specs=pl.BlockSpec((b, ft), lambda i: (0, i)),
        out_shape=jax.ShapeDtypeStruct((b, f), jnp.float32),
    )(x, w_enc, b_enc.reshape(1, f), b_dec.reshape(1, d))


_NBUK_A = 32768
_ROWS_PER_W = 8
_CCOLS = 2048


def _sc_mesh():
    return plsc.VectorSubcoreMesh(core_axis_name="c", subcore_axis_name="s")


def _zero_ref(ref, nwords):
    zeros = jnp.zeros((_LANES,), jnp.int32)

    def zbody(i, carry):
        ref[pl.ds(i * _LANES, _LANES)] = zeros
        return carry

    lax.fori_loop(0, nwords // _LANES, zbody, 0)


def _scan_chunks(acts_hbm, row0, ncols, buf0, buf1, sem0, sem1, process):
    n_chunks = ncols // _CCOLS

    def copy(c, buf, sem):
        return pltpu.make_async_copy(
            acts_hbm.at[pl.ds(row0, _ROWS_PER_W), pl.ds(c * _CCOLS, _CCOLS)],
            buf, sem)

    copy(0, buf0, sem0).start()

    def cbody(i, carry):
        for par, (buf_a, sem_a, buf_b, sem_b) in (
                (0, (buf0, sem0, buf1, sem1)),
                (1, (buf1, sem1, buf0, sem0))):
            c = i * 2 + par

            @pl.when(c + 1 < n_chunks)
            def _():
                copy(c + 1, buf_b, sem_b).start()

            copy(c, buf_a, sem_a).wait()
            process(buf_a)
        return carry

    lax.fori_loop(0, n_chunks // 2, cbody, 0)


def _make_hist_a(b, f):
    info = plsc.get_sparse_core_info()
    nw = info.num_cores * info.num_subcores

    @functools.partial(
        pl.kernel,
        mesh=_sc_mesh(),
        out_type=jax.ShapeDtypeStruct((nw, 2, _NBUK_A), jnp.int32),
        compiler_params=pltpu.CompilerParams(needs_layout_passes=False),
        scratch_types=[
            pltpu.VMEM((_ROWS_PER_W, _CCOLS), jnp.float32),
            pltpu.VMEM((_ROWS_PER_W, _CCOLS), jnp.float32),
            pltpu.VMEM((_NBUK_A,), jnp.int32),
            pltpu.VMEM((_NBUK_A,), jnp.int32),
            pltpu.SemaphoreType.DMA,
            pltpu.SemaphoreType.DMA,
        ],
    )
    def hist_a(acts_hbm, out_hbm, buf0, buf1, h0, h1, sem0, sem1):
        wid = lax.axis_index("s") * info.num_cores + lax.axis_index("c")
        ones = jnp.ones((_LANES,), jnp.int32)
        zerosv = jnp.zeros((_LANES,), jnp.int32)
        _zero_ref(h0, _NBUK_A)
        _zero_ref(h1, _NBUK_A)

        def process(buf):
            def vbody(j, carry):
                for r in range(_ROWS_PER_W):
                    v = buf[r, pl.ds(j * _LANES, _LANES)]
                    bits = plsc.bitcast(v, jnp.int32)
                    buk = lax.shift_right_logical(bits, 16)
                    hist = h0 if r % 2 == 0 else h1
                    plsc.addupdate_scatter(hist, [buk], ones,
                                           mask=bits != zerosv)
                return carry

            lax.fori_loop(0, _CCOLS // _LANES, vbody, 0)

        _scan_chunks(acts_hbm, wid * _ROWS_PER_W, f, buf0, buf1, sem0, sem1,
                     process)
        pltpu.sync_copy(h0, out_hbm.at[wid, 0])
        pltpu.sync_copy(h1, out_hbm.at[wid, 1])

    return hist_a


def _make_hist_b(b, f):
    info = plsc.get_sparse_core_info()
    nw = info.num_cores * info.num_subcores

    @functools.partial(
        pl.kernel,
        mesh=_sc_mesh(),
        out_type=jax.ShapeDtypeStruct((nw, _NBUK), jnp.int32),
        compiler_params=pltpu.CompilerParams(needs_layout_passes=False),
        scratch_types=[
            pltpu.VMEM((_ROWS_PER_W, _CCOLS), jnp.float32),
            pltpu.VMEM((_ROWS_PER_W, _CCOLS), jnp.float32),
            pltpu.VMEM((_NBUK,), jnp.int32),
            pltpu.VMEM((_LANES,), jnp.int32),
            pltpu.SemaphoreType.DMA,
            pltpu.SemaphoreType.DMA,
        ],
    )
    def hist_b(acts_hbm, hstar_hbm, out_hbm, buf0, buf1, h0, hsv, sem0, sem1):
        wid = lax.axis_index("s") * info.num_cores + lax.axis_index("c")
        pltpu.sync_copy(hstar_hbm, hsv)
        h16 = hsv[...]
        ones = jnp.ones((_LANES,), jnp.int32)
        lowmask = jnp.full((_LANES,), 0xFFFF, jnp.int32)
        _zero_ref(h0, _NBUK)

        def process(buf):
            def vbody(j, carry):
                for r in range(_ROWS_PER_W):
                    v = buf[r, pl.ds(j * _LANES, _LANES)]
                    bits = plsc.bitcast(v, jnp.int32)
                    match = lax.shift_right_logical(bits, 16) == h16
                    buk = bits & lowmask
                    plsc.addupdate_scatter(h0, [buk], ones, mask=match)
                return carry

            lax.fori_loop(0, _CCOLS // _LANES, vbody, 0)

        _scan_chunks(acts_hbm, wid * _ROWS_PER_W, f, buf0, buf1, sem0, sem1,
                     process)
        pltpu.sync_copy(h0, out_hbm.at[wid])

    return hist_b


def _rank_threshold(acts, kb):
    b, f = acts.shape
    cnt_a = _make_hist_a(b, f)(acts).sum(axis=(0, 1))
    t_a = jnp.cumsum(cnt_a[::-1])[::-1]
    hstar = jnp.sum(t_a >= kb).astype(jnp.int32) - 1
    t_next = jnp.concatenate([t_a[1:], jnp.zeros((1,), t_a.dtype)])
    r = kb - t_next[jnp.maximum(hstar, 0)]

    h16 = jnp.full((_LANES,), jnp.maximum(hstar, 0), jnp.int32)
    cnt_b = _make_hist_b(b, f)(acts, h16).sum(axis=0)
    t_b = jnp.cumsum(cnt_b[::-1])[::-1]
    lstar = jnp.sum(t_b >= r).astype(jnp.int32) - 1

    vk_bits = jnp.left_shift(jnp.maximum(hstar, 0), 16) | jnp.maximum(lstar, 0)
    vk = lax.bitcast_convert_type(vk_bits, jnp.float32)
    return jnp.where(hstar < 0, jnp.float32(0.0), vk)


def _decode_body(vk_ref, acts_ref, w_ref, bd_ref, out_ref):
    i = pl.program_id(0)
    vk = vk_ref[0, 0]
    a = acts_ref[...]
    m = jnp.where(a >= vk, a, 0.0)
    part = lax.dot_general(m, w_ref[...], (((1,), (1,)), ((), ())),
                           preferred_element_type=jnp.float32)

    @pl.when(i == 0)
    def _():
        out_ref[...] = jnp.broadcast_to(bd_ref[...], out_ref.shape)

    out_ref[...] += part


def _decode(vk, acts, w_dec, b_dec, ft):
    b, f = acts.shape
    d = w_dec.shape[0]
    grid = (f // ft,)
    return pl.pallas_call(
        _decode_body,
        grid=grid,
        in_specs=[
            pl.BlockSpec(memory_space=pltpu.SMEM),
            pl.BlockSpec((b, ft), lambda i: (0, i)),
            pl.BlockSpec((d, ft), lambda i: (0, i)),
            pl.BlockSpec((1, d), lambda i: (0, 0)),
        ],
        out_specs=pl.BlockSpec((b, d), lambda i: (0, 0)),
        out_shape=jax.ShapeDtypeStruct((b, d), jnp.float32),
    )(vk.reshape(1, 1), acts, w_dec, b_dec.reshape(1, d))


def kernel(x, W_enc, b_enc, W_dec, b_dec):
    b, d = x.shape
    f = W_enc.shape[0]
    kb = min(64 * b, b * f)
    acts = _encode(x, W_enc, b_enc, b_dec, ft=2048)
    vk = _rank_threshold(acts, kb)
    return _decode(vk, acts, W_dec, b_dec, ft=2048)


# speedup vs baseline: 37.9250x; 1.8387x over previous
import functools

import jax
import jax.numpy as jnp
from jax import lax
from jax.experimental import pallas as pl
from jax.experimental.pallas import tpu as pltpu
from jax.experimental.pallas import tpu_sc as plsc

_NBUK = 65536
_LANES = 16


def _encode_body(x_ref, w_ref, be_ref, bd_ref, acts_ref):
    xm = x_ref[...] - bd_ref[...]
    a = lax.dot_general(xm, w_ref[...], (((1,), (1,)), ((), ())),
                        preferred_element_type=jnp.float32)
    acts_ref[...] = jnp.maximum(a + be_ref[...], 0.0)


def _encode(x, w_enc, b_enc, b_dec, ft):
    b, d = x.shape
    f = w_enc.shape[0]
    grid = (f // ft,)
    return pl.pallas_call(
        _encode_body,
        grid=grid,
        in_specs=[
            pl.BlockSpec((b, d), lambda i: (0, 0)),
            pl.BlockSpec((ft, d), lambda i: (i, 0)),
            pl.BlockSpec((1, ft), lambda i: (0, i)),
            pl.BlockSpec((1, d), lambda i: (0, 0)),
        ],
        out_specs=pl.BlockSpec((b, ft), lambda i: (0, i)),
        out_shape=jax.ShapeDtypeStruct((b, f), jnp.float32),
    )(x, w_enc, b_enc.reshape(1, f), b_dec.reshape(1, d))


_NBUK_A = 32768
_ROWS_PER_W = 8
_CCOLS = 2048


def _sc_mesh():
    return plsc.VectorSubcoreMesh(core_axis_name="c", subcore_axis_name="s")


def _zero_ref(ref, nwords):
    zeros = jnp.zeros((_LANES,), jnp.int32)

    @plsc.parallel_loop(0, nwords // _LANES, unroll=8)
    def _(i):
        ref[pl.ds(i * _LANES, _LANES)] = zeros


def _scan_chunks(acts_hbm, row0, ncols, buf0, buf1, sem0, sem1, process):
    n_chunks = ncols // _CCOLS

    def copy(c, buf, sem):
        return pltpu.make_async_copy(
            acts_hbm.at[pl.ds(row0, _ROWS_PER_W), pl.ds(c * _CCOLS, _CCOLS)],
            buf, sem)

    copy(0, buf0, sem0).start()

    def cbody(i, carry):
        for par, (buf_a, sem_a, buf_b, sem_b) in (
                (0, (buf0, sem0, buf1, sem1)),
                (1, (buf1, sem1, buf0, sem0))):
            c = i * 2 + par

            @pl.when(c + 1 < n_chunks)
            def _():
                copy(c + 1, buf_b, sem_b).start()

            copy(c, buf_a, sem_a).wait()
            process(buf_a)
        return carry

    lax.fori_loop(0, n_chunks // 2, cbody, 0)


def _make_hist_a(b, f):
    info = plsc.get_sparse_core_info()
    nw = info.num_cores * info.num_subcores

    @functools.partial(
        pl.kernel,
        mesh=_sc_mesh(),
        out_type=jax.ShapeDtypeStruct((nw, 2, _NBUK_A), jnp.int32),
        compiler_params=pltpu.CompilerParams(needs_layout_passes=False),
        scratch_types=[
            pltpu.VMEM((_ROWS_PER_W, _CCOLS), jnp.float32),
            pltpu.VMEM((_ROWS_PER_W, _CCOLS), jnp.float32),
            pltpu.VMEM((_NBUK_A,), jnp.int32),
            pltpu.VMEM((_NBUK_A,), jnp.int32),
            pltpu.SemaphoreType.DMA,
            pltpu.SemaphoreType.DMA,
        ],
    )
    def hist_a(acts_hbm, out_hbm, buf0, buf1, h0, h1, sem0, sem1):
        wid = lax.axis_index("s") * info.num_cores + lax.axis_index("c")
        ones = jnp.ones((_LANES,), jnp.int32)
        zerosv = jnp.zeros((_LANES,), jnp.int32)
        _zero_ref(h0, _NBUK_A)
        _zero_ref(h1, _NBUK_A)

        def process(buf):
            @plsc.parallel_loop(0, _CCOLS // _LANES, unroll=4)
            def _(j):
                for r in range(_ROWS_PER_W):
                    v = buf[r, pl.ds(j * _LANES, _LANES)]
                    bits = plsc.bitcast(v, jnp.int32)
                    buk = lax.shift_right_logical(bits, 16)
                    hist = h0 if r % 2 == 0 else h1
                    plsc.addupdate_scatter(hist, [buk], ones,
                                           mask=bits != zerosv)

        _scan_chunks(acts_hbm, wid * _ROWS_PER_W, f, buf0, buf1, sem0, sem1,
                     process)
        pltpu.sync_copy(h0, out_hbm.at[wid, 0])
        pltpu.sync_copy(h1, out_hbm.at[wid, 1])

    return hist_a


def _make_hist_b(b, f):
    info = plsc.get_sparse_core_info()
    nw = info.num_cores * info.num_subcores

    @functools.partial(
        pl.kernel,
        mesh=_sc_mesh(),
        out_type=jax.ShapeDtypeStruct((nw, _NBUK), jnp.int32),
        compiler_params=pltpu.CompilerParams(needs_layout_passes=False),
        scratch_types=[
            pltpu.VMEM((_ROWS_PER_W, _CCOLS), jnp.float32),
            pltpu.VMEM((_ROWS_PER_W, _CCOLS), jnp.float32),
            pltpu.VMEM((_NBUK,), jnp.int32),
            pltpu.VMEM((_LANES,), jnp.int32),
            pltpu.SemaphoreType.DMA,
            pltpu.SemaphoreType.DMA,
        ],
    )
    def hist_b(acts_hbm, hstar_hbm, out_hbm, buf0, buf1, h0, hsv, sem0, sem1):
        wid = lax.axis_index("s") * info.num_cores + lax.axis_index("c")
        pltpu.sync_copy(hstar_hbm, hsv)
        h16 = hsv[...]
        ones = jnp.ones((_LANES,), jnp.int32)
        lowmask = jnp.full((_LANES,), 0xFFFF, jnp.int32)
        _zero_ref(h0, _NBUK)

        def process(buf):
            @plsc.parallel_loop(0, _CCOLS // _LANES, unroll=4)
            def _(j):
                for r in range(_ROWS_PER_W):
                    v = buf[r, pl.ds(j * _LANES, _LANES)]
                    bits = plsc.bitcast(v, jnp.int32)
                    match = lax.shift_right_logical(bits, 16) == h16
                    buk = bits & lowmask
                    plsc.addupdate_scatter(h0, [buk], ones, mask=match)

        _scan_chunks(acts_hbm, wid * _ROWS_PER_W, f, buf0, buf1, sem0, sem1,
                     process)
        pltpu.sync_copy(h0, out_hbm.at[wid])

    return hist_b


def _rank_threshold(acts, kb):
    b, f = acts.shape
    cnt_a = _make_hist_a(b, f)(acts).sum(axis=(0, 1))
    t_a = jnp.cumsum(cnt_a[::-1])[::-1]
    hstar = jnp.sum(t_a >= kb).astype(jnp.int32) - 1
    t_next = jnp.concatenate([t_a[1:], jnp.zeros((1,), t_a.dtype)])
    r = kb - t_next[jnp.maximum(hstar, 0)]

    h16 = jnp.full((_LANES,), jnp.maximum(hstar, 0), jnp.int32)
    cnt_b = _make_hist_b(b, f)(acts, h16).sum(axis=0)
    t_b = jnp.cumsum(cnt_b[::-1])[::-1]
    lstar = jnp.sum(t_b >= r).astype(jnp.int32) - 1

    vk_bits = jnp.left_shift(jnp.maximum(hstar, 0), 16) | jnp.maximum(lstar, 0)
    vk = lax.bitcast_convert_type(vk_bits, jnp.float32)
    return jnp.where(hstar < 0, jnp.float32(0.0), vk)


def _decode_body(vk_ref, acts_ref, w_ref, bd_ref, out_ref):
    i = pl.program_id(0)
    vk = vk_ref[0, 0]
    a = acts_ref[...]
    m = jnp.where(a >= vk, a, 0.0)
    part = lax.dot_general(m, w_ref[...], (((1,), (1,)), ((), ())),
                           preferred_element_type=jnp.float32)

    @pl.when(i == 0)
    def _():
        out_ref[...] = jnp.broadcast_to(bd_ref[...], out_ref.shape)

    out_ref[...] += part


def _decode(vk, acts, w_dec, b_dec, ft):
    b, f = acts.shape
    d = w_dec.shape[0]
    grid = (f // ft,)
    return pl.pallas_call(
        _decode_body,
        grid=grid,
        in_specs=[
            pl.BlockSpec(memory_space=pltpu.SMEM),
            pl.BlockSpec((b, ft), lambda i: (0, i)),
            pl.BlockSpec((d, ft), lambda i: (0, i)),
            pl.BlockSpec((1, d), lambda i: (0, 0)),
        ],
        out_specs=pl.BlockSpec((b, d), lambda i: (0, 0)),
        out_shape=jax.ShapeDtypeStruct((b, d), jnp.float32),
    )(vk.reshape(1, 1), acts, w_dec, b_dec.reshape(1, d))


def kernel(x, W_enc, b_enc, W_dec, b_dec):
    b, d = x.shape
    f = W_enc.shape[0]
    kb = min(64 * b, b * f)
    acts = _encode(x, W_enc, b_enc, b_dec, ft=2048)
    vk = _rank_threshold(acts, kb)
    return _decode(vk, acts, W_dec, b_dec, ft=2048)
